# Initial kernel scaffold; baseline (speedup 1.0000x reference)
import functools

import jax
import jax.numpy as jnp
from jax import lax
from jax.experimental import pallas as pl
from jax.experimental.pallas import tpu as pltpu
from jax.experimental.pallas import tpu_sc as plsc

NC = 2
NS = 16
NW = NC * NS
CH = 128

_MESH = plsc.VectorSubcoreMesh(core_axis_name="c", subcore_axis_name="s")
_PREC = jax.lax.Precision.HIGHEST


def _deg_kernel_factory(rows_w, np_, wcols):
    @functools.partial(
        pl.kernel,
        mesh=_MESH,
        out_type=jax.ShapeDtypeStruct((NC, np_, wcols), jnp.float32),
        scratch_types=[
            pltpu.VMEM((rows_w, CH), jnp.int32),
            pltpu.VMEM((CH, wcols), jnp.float32),
            pltpu.VMEM_SHARED((np_, wcols), jnp.float32),
        ],
    )
    def deg_kernel(dst_hbm, ones_hbm, zeros_hbm, out_hbm, idx_v, ones_v, acc):
        c = lax.axis_index("c")
        s = lax.axis_index("s")
        w = s * NC + c
        rpz = np_ // NS
        r0 = s * rpz
        pltpu.sync_copy(zeros_hbm.at[pl.ds(r0, rpz)], acc.at[pl.ds(r0, rpz)])
        pltpu.sync_copy(ones_hbm, ones_v)
        pltpu.sync_copy(dst_hbm.at[pl.ds(w * rows_w, rows_w)], idx_v)
        plsc.subcore_barrier()

        @pl.loop(0, rows_w)
        def _(j):
            pltpu.sync_copy(ones_v, acc.at[idx_v.at[j]], add=True)

        plsc.subcore_barrier()
        pltpu.sync_copy(acc.at[pl.ds(r0, rpz)], out_hbm.at[c, pl.ds(r0, rpz)])

    return deg_kernel


def _agg_kernel_factory(rows_w, np_, h):
    @functools.partial(
        pl.kernel,
        mesh=_MESH,
        out_type=jax.ShapeDtypeStruct((NC, np_, h), jnp.float32),
        scratch_types=[
            pltpu.VMEM((rows_w, CH), jnp.int32),
            pltpu.VMEM((rows_w, CH), jnp.int32),
            pltpu.VMEM((CH, h), jnp.float32),
            pltpu.VMEM_SHARED((np_, h), jnp.float32),
        ],
    )
    def agg_kernel(hp_hbm, src_hbm, dst_hbm, zeros_hbm, out_hbm,
                   src_v, dst_v, msg_v, acc):
        c = lax.axis_index("c")
        s = lax.axis_index("s")
        w = s * NC + c
        rpz = np_ // NS
        r0 = s * rpz
        pltpu.sync_copy(zeros_hbm.at[pl.ds(r0, rpz)], acc.at[pl.ds(r0, rpz)])
        pltpu.sync_copy(src_hbm.at[pl.ds(w * rows_w, rows_w)], src_v)
        pltpu.sync_copy(dst_hbm.at[pl.ds(w * rows_w, rows_w)], dst_v)
        plsc.subcore_barrier()

        @pl.loop(0, rows_w)
        def _(j):
            pltpu.sync_copy(hp_hbm.at[src_v.at[j]], msg_v)
            pltpu.sync_copy(msg_v, acc.at[dst_v.at[j]], add=True)

        plsc.subcore_barrier()
        pltpu.sync_copy(acc.at[pl.ds(r0, rpz)], out_hbm.at[c, pl.ds(r0, rpz)])

    return agg_kernel


def _matmul(x, w, br):
    n, d = x.shape
    h = w.shape[1]

    def body(x_ref, w_ref, o_ref):
        o_ref[...] = lax.dot_general(
            x_ref[...], w_ref[...], (((1,), (0,)), ((), ())),
            precision=_PREC, preferred_element_type=jnp.float32)

    return pl.pallas_call(
        body,
        grid=(n // br,),
        in_specs=[
            pl.BlockSpec((br, d), lambda i: (i, 0)),
            pl.BlockSpec((d, h), lambda i: (0, 0)),
        ],
        out_specs=pl.BlockSpec((br, h), lambda i: (i, 0)),
        out_shape=jax.ShapeDtypeStruct((n, h), jnp.float32),
    )(x, w)


def _deg_stats(dp_ref):
    deg = 1.0 + dp_ref[0, :, 0:1] + dp_ref[1, :, 0:1]
    return lax.rsqrt(deg), 1.0 / deg


def _scale(deg_parts, h1, br, wcols):
    n, h = h1.shape

    def body(dp_ref, h_ref, o_ref):
        dis, _ = _deg_stats(dp_ref)
        o_ref[...] = h_ref[...] * dis

    return pl.pallas_call(
        body,
        grid=(n // br,),
        in_specs=[
            pl.BlockSpec((NC, br, wcols), lambda i: (0, i, 0)),
            pl.BlockSpec((br, h), lambda i: (i, 0)),
        ],
        out_specs=pl.BlockSpec((br, h), lambda i: (i, 0)),
        out_shape=jax.ShapeDtypeStruct((n, h), jnp.float32),
    )(deg_parts, h1)


def _layer2(parts1, deg_parts, h1, b1, w2, br, wcols):
    n, h = h1.shape
    h2w = w2.shape[1]

    def body(p_ref, dp_ref, h1_ref, b1_ref, w2_ref, h2_ref, hp2_ref):
        dis, invd = _deg_stats(dp_ref)
        out1 = dis * (p_ref[0] + p_ref[1]) + h1_ref[...] * invd + b1_ref[...]
        a1 = jnp.maximum(out1, 0.0)
        h2 = lax.dot_general(a1, w2_ref[...], (((1,), (0,)), ((), ())),
                             precision=_PREC, preferred_element_type=jnp.float32)
        h2_ref[...] = h2
        hp2_ref[...] = h2 * dis

    return pl.pallas_call(
        body,
        grid=(n // br,),
        in_specs=[
            pl.BlockSpec((NC, br, h), lambda i: (0, i, 0)),
            pl.BlockSpec((NC, br, wcols), lambda i: (0, i, 0)),
            pl.BlockSpec((br, h), lambda i: (i, 0)),
            pl.BlockSpec((1, h), lambda i: (0, 0)),
            pl.BlockSpec((h, h2w), lambda i: (0, 0)),
        ],
        out_specs=[
            pl.BlockSpec((br, h2w), lambda i: (i, 0)),
            pl.BlockSpec((br, h2w), lambda i: (i, 0)),
        ],
        out_shape=[
            jax.ShapeDtypeStruct((n, h2w), jnp.float32),
            jax.ShapeDtypeStruct((n, h2w), jnp.float32),
        ],
    )(parts1, deg_parts, h1, b1, w2)


def _final(parts2, deg_parts, h2, b2, br, wcols):
    n, h = h2.shape

    def body(p_ref, dp_ref, h2_ref, b2_ref, o_ref):
        dis, invd = _deg_stats(dp_ref)
        o_ref[...] = (dis * (p_ref[0] + p_ref[1])
                      + h2_ref[...] * invd + b2_ref[...])

    return pl.pallas_call(
        body,
        grid=(n // br,),
        in_specs=[
            pl.BlockSpec((NC, br, h), lambda i: (0, i, 0)),
            pl.BlockSpec((NC, br, wcols), lambda i: (0, i, 0)),
            pl.BlockSpec((br, h), lambda i: (i, 0)),
            pl.BlockSpec((1, h), lambda i: (0, 0)),
        ],
        out_specs=pl.BlockSpec((br, h), lambda i: (i, 0)),
        out_shape=jax.ShapeDtypeStruct((n, h), jnp.float32),
    )(parts2, deg_parts, h2, b2)


def kernel(x, edge_index, W1, b1, W2, b2):
    n, d = x.shape
    e = edge_index.shape[1]
    h1w = W1.shape[1]
    h2w = W2.shape[1]

    br = 1024
    np_ = ((n + br - 1) // br) * br
    wcols = 16
    rows_w = -(-e // (CH * NW))
    e_pad = rows_w * NW * CH

    sent = jnp.int32(n)
    pad = jnp.full((e_pad - e,), sent, jnp.int32)
    src2d = jnp.concatenate([edge_index[0], pad]).reshape(NW * rows_w, CH)
    dst2d = jnp.concatenate([edge_index[1], pad]).reshape(NW * rows_w, CH)

    x_pad = jnp.pad(x, ((0, np_ - n), (0, 0)))
    ones_img = jnp.ones((CH, wcols), jnp.float32)
    zeros_w = jnp.zeros((np_, wcols), jnp.float32)
    zeros_h1 = jnp.zeros((np_, h1w), jnp.float32)
    zeros_h2 = jnp.zeros((np_, h2w), jnp.float32)

    deg_parts = _deg_kernel_factory(rows_w, np_, wcols)(dst2d, ones_img, zeros_w)
    h1 = _matmul(x_pad, W1, br)
    hp1 = _scale(deg_parts, h1, br, wcols)
    parts1 = _agg_kernel_factory(rows_w, np_, h1w)(hp1, src2d, dst2d, zeros_h1)
    h2, hp2 = _layer2(parts1, deg_parts, h1, b1.reshape(1, h1w), W2, br, wcols)
    parts2 = _agg_kernel_factory(rows_w, np_, h2w)(hp2, src2d, dst2d, zeros_h2)
    out = _final(parts2, deg_parts, h2, b2.reshape(1, h2w), br, wcols)
    return out[:n]


# speedup vs baseline: 25.1596x; 25.1596x over previous
import functools

import jax
import jax.numpy as jnp
from jax import lax
from jax.experimental import pallas as pl
from jax.experimental.pallas import tpu as pltpu
from jax.experimental.pallas import tpu_sc as plsc

NC = 2
NS = 16
NW = NC * NS
CH = 128

_MESH = plsc.VectorSubcoreMesh(core_axis_name="c", subcore_axis_name="s")
_PREC = jax.lax.Precision.HIGHEST


def _deg_kernel_factory(rows_w, np_, wcols):
    @functools.partial(
        pl.kernel,
        mesh=_MESH,
        out_type=jax.ShapeDtypeStruct((NC, np_, wcols), jnp.float32),
        scratch_types=[
            pltpu.VMEM((rows_w, CH), jnp.int32),
            pltpu.VMEM((CH, wcols), jnp.float32),
            pltpu.VMEM_SHARED((np_, wcols), jnp.float32),
        ],
    )
    def deg_kernel(dst_hbm, ones_hbm, zeros_hbm, out_hbm, idx_v, ones_v, acc):
        c = lax.axis_index("c")
        s = lax.axis_index("s")
        w = s * NC + c
        rpz = np_ // NS
        r0 = s * rpz
        pltpu.sync_copy(zeros_hbm.at[pl.ds(r0, rpz)], acc.at[pl.ds(r0, rpz)])
        pltpu.sync_copy(ones_hbm, ones_v)
        pltpu.sync_copy(dst_hbm.at[pl.ds(w * rows_w, rows_w)], idx_v)
        plsc.subcore_barrier()

        @pl.loop(0, rows_w)
        def _(j):
            pltpu.sync_copy(ones_v, acc.at[idx_v.at[j]], add=True)

        plsc.subcore_barrier()
        pltpu.sync_copy(acc.at[pl.ds(r0, rpz)], out_hbm.at[c, pl.ds(r0, rpz)])

    return deg_kernel


def _agg_kernel_factory(rows_w, np_, h):
    @functools.partial(
        pl.kernel,
        mesh=_MESH,
        out_type=jax.ShapeDtypeStruct((NC, np_, h), jnp.float32),
        scratch_types=[
            pltpu.VMEM((rows_w, CH), jnp.int32),
            pltpu.VMEM((rows_w, CH), jnp.int32),
            pltpu.VMEM((CH, h), jnp.float32),
            pltpu.VMEM_SHARED((np_, h), jnp.float32),
        ],
        compiler_params=pltpu.CompilerParams(use_tc_tiling_on_sc=False),
    )
    def agg_kernel(hp_hbm, src_hbm, dst_hbm, zeros_hbm, out_hbm,
                   src_v, dst_v, msg_v, acc):
        c = lax.axis_index("c")
        s = lax.axis_index("s")
        w = s * NC + c
        rpz = np_ // NS
        r0 = s * rpz
        pltpu.sync_copy(zeros_hbm.at[pl.ds(r0, rpz)], acc.at[pl.ds(r0, rpz)])
        pltpu.sync_copy(src_hbm.at[pl.ds(w * rows_w, rows_w)], src_v)
        pltpu.sync_copy(dst_hbm.at[pl.ds(w * rows_w, rows_w)], dst_v)
        plsc.subcore_barrier()

        @pl.loop(0, rows_w)
        def _(j):
            pltpu.sync_copy(hp_hbm.at[src_v.at[j]], msg_v)
            pltpu.sync_copy(msg_v, acc.at[dst_v.at[j]], add=True)

        plsc.subcore_barrier()
        pltpu.sync_copy(acc.at[pl.ds(r0, rpz)], out_hbm.at[c, pl.ds(r0, rpz)])

    return agg_kernel


def _matmul(x, w, br):
    n, d = x.shape
    h = w.shape[1]

    def body(x_ref, w_ref, o_ref):
        o_ref[...] = lax.dot_general(
            x_ref[...], w_ref[...], (((1,), (0,)), ((), ())),
            precision=_PREC, preferred_element_type=jnp.float32)

    return pl.pallas_call(
        body,
        grid=(n // br,),
        in_specs=[
            pl.BlockSpec((br, d), lambda i: (i, 0)),
            pl.BlockSpec((d, h), lambda i: (0, 0)),
        ],
        out_specs=pl.BlockSpec((br, h), lambda i: (i, 0)),
        out_shape=jax.ShapeDtypeStruct((n, h), jnp.float32),
    )(x, w)


def _deg_stats(dp_ref):
    deg = 1.0 + dp_ref[0, :, 0:1] + dp_ref[1, :, 0:1]
    return lax.rsqrt(deg), 1.0 / deg


def _scale(deg_parts, h1, br, wcols):
    n, h = h1.shape

    def body(dp_ref, h_ref, o_ref):
        dis, _ = _deg_stats(dp_ref)
        o_ref[...] = h_ref[...] * dis

    return pl.pallas_call(
        body,
        grid=(n // br,),
        in_specs=[
            pl.BlockSpec((NC, br, wcols), lambda i: (0, i, 0)),
            pl.BlockSpec((br, h), lambda i: (i, 0)),
        ],
        out_specs=pl.BlockSpec((br, h), lambda i: (i, 0)),
        out_shape=jax.ShapeDtypeStruct((n, h), jnp.float32),
    )(deg_parts, h1)


def _layer2(parts1, deg_parts, h1, b1, w2, br, wcols):
    n, h = h1.shape
    h2w = w2.shape[1]

    def body(p_ref, dp_ref, h1_ref, b1_ref, w2_ref, h2_ref, hp2_ref):
        dis, invd = _deg_stats(dp_ref)
        out1 = dis * (p_ref[0] + p_ref[1]) + h1_ref[...] * invd + b1_ref[...]
        a1 = jnp.maximum(out1, 0.0)
        h2 = lax.dot_general(a1, w2_ref[...], (((1,), (0,)), ((), ())),
                             precision=_PREC, preferred_element_type=jnp.float32)
        h2_ref[...] = h2
        hp2_ref[...] = h2 * dis

    return pl.pallas_call(
        body,
        grid=(n // br,),
        in_specs=[
            pl.BlockSpec((NC, br, h), lambda i: (0, i, 0)),
            pl.BlockSpec((NC, br, wcols), lambda i: (0, i, 0)),
            pl.BlockSpec((br, h), lambda i: (i, 0)),
            pl.BlockSpec((1, h), lambda i: (0, 0)),
            pl.BlockSpec((h, h2w), lambda i: (0, 0)),
        ],
        out_specs=[
            pl.BlockSpec((br, h2w), lambda i: (i, 0)),
            pl.BlockSpec((br, h2w), lambda i: (i, 0)),
        ],
        out_shape=[
            jax.ShapeDtypeStruct((n, h2w), jnp.float32),
            jax.ShapeDtypeStruct((n, h2w), jnp.float32),
        ],
    )(parts1, deg_parts, h1, b1, w2)


def _final(parts2, deg_parts, h2, b2, br, wcols):
    n, h = h2.shape

    def body(p_ref, dp_ref, h2_ref, b2_ref, o_ref):
        dis, invd = _deg_stats(dp_ref)
        o_ref[...] = (dis * (p_ref[0] + p_ref[1])
                      + h2_ref[...] * invd + b2_ref[...])

    return pl.pallas_call(
        body,
        grid=(n // br,),
        in_specs=[
            pl.BlockSpec((NC, br, h), lambda i: (0, i, 0)),
            pl.BlockSpec((NC, br, wcols), lambda i: (0, i, 0)),
            pl.BlockSpec((br, h), lambda i: (i, 0)),
            pl.BlockSpec((1, h), lambda i: (0, 0)),
        ],
        out_specs=pl.BlockSpec((br, h), lambda i: (i, 0)),
        out_shape=jax.ShapeDtypeStruct((n, h), jnp.float32),
    )(parts2, deg_parts, h2, b2)


def kernel(x, edge_index, W1, b1, W2, b2):
    n, d = x.shape
    e = edge_index.shape[1]
    h1w = W1.shape[1]
    h2w = W2.shape[1]

    br = 1024
    np_ = ((n + br - 1) // br) * br
    wcols = 16
    rows_w = -(-e // (CH * NW))
    rows_w = ((rows_w + 7) // 8) * 8
    e_pad = rows_w * NW * CH

    sent = jnp.int32(n)
    pad = jnp.full((e_pad - e,), sent, jnp.int32)
    src2d = jnp.concatenate([edge_index[0], pad]).reshape(NW * rows_w, CH)
    dst2d = jnp.concatenate([edge_index[1], pad]).reshape(NW * rows_w, CH)

    x_pad = jnp.pad(x, ((0, np_ - n), (0, 0)))
    ones_img = jnp.ones((CH, wcols), jnp.float32)
    zeros_w = jnp.zeros((np_, wcols), jnp.float32)
    zeros_h1 = jnp.zeros((np_, h1w), jnp.float32)
    zeros_h2 = jnp.zeros((np_, h2w), jnp.float32)

    deg_parts = _deg_kernel_factory(rows_w, np_, wcols)(dst2d, ones_img, zeros_w)
    h1 = _matmul(x_pad, W1, br)
    hp1 = _scale(deg_parts, h1, br, wcols)
    parts1 = _agg_kernel_factory(rows_w, np_, h1w)(hp1, src2d, dst2d, zeros_h1)
    h2, hp2 = _layer2(parts1, deg_parts, h1, b1.reshape(1, h1w), W2, br, wcols)
    parts2 = _agg_kernel_factory(rows_w, np_, h2w)(hp2, src2d, dst2d, zeros_h2)
    out = _final(parts2, deg_parts, h2, b2.reshape(1, h2w), br, wcols)
    return out[:n]


# speedup vs baseline: 31.2873x; 1.2436x over previous
import functools

import jax
import jax.numpy as jnp
from jax import lax
from jax.experimental import pallas as pl
from jax.experimental.pallas import tpu as pltpu
from jax.experimental.pallas import tpu_sc as plsc

NC = 2
NS = 16
NW = NC * NS
CH = 128

_MESH = plsc.VectorSubcoreMesh(core_axis_name="c", subcore_axis_name="s")
_PREC = jax.lax.Precision.HIGHEST


def _deg_kernel_factory(rows_w, np_, wcols):
    @functools.partial(
        pl.kernel,
        mesh=_MESH,
        out_type=jax.ShapeDtypeStruct((NC, np_, wcols), jnp.float32),
        scratch_types=[
            pltpu.VMEM((rows_w, CH), jnp.int32),
            pltpu.VMEM((CH, wcols), jnp.float32),
            pltpu.VMEM_SHARED((np_, wcols), jnp.float32),
            pltpu.SemaphoreType.DMA,
        ],
    )
    def deg_kernel(dst_hbm, ones_hbm, zeros_hbm, out_hbm, idx_v, ones_v, acc,
                   ssem):
        c = lax.axis_index("c")
        s = lax.axis_index("s")
        w = s * NC + c
        rpz = np_ // NS
        r0 = s * rpz
        pltpu.sync_copy(zeros_hbm.at[pl.ds(r0, rpz)], acc.at[pl.ds(r0, rpz)])
        pltpu.sync_copy(ones_hbm, ones_v)
        pltpu.sync_copy(dst_hbm.at[pl.ds(w * rows_w, rows_w)], idx_v)
        plsc.subcore_barrier()

        @pl.loop(0, rows_w, step=8)
        def _(j):
            for b in range(8):
                pltpu.async_copy(ones_v, acc.at[idx_v.at[j + b]], ssem,
                                 add=True)
            for b in range(8):
                pltpu.make_async_copy(ones_v, acc.at[idx_v.at[j + b]],
                                      ssem).wait()

        plsc.subcore_barrier()
        pltpu.sync_copy(acc.at[pl.ds(r0, rpz)], out_hbm.at[c, pl.ds(r0, rpz)])

    return deg_kernel


def _agg_kernel_factory(rows_w, np_, h):
    @functools.partial(
        pl.kernel,
        mesh=_MESH,
        out_type=jax.ShapeDtypeStruct((NC, np_, h), jnp.float32),
        scratch_types=[
            pltpu.VMEM((rows_w, CH), jnp.int32),
            pltpu.VMEM((rows_w, CH), jnp.int32),
            pltpu.VMEM((CH, h), jnp.float32),
            pltpu.VMEM((CH, h), jnp.float32),
            pltpu.VMEM_SHARED((np_, h), jnp.float32),
            pltpu.SemaphoreType.DMA,
            pltpu.SemaphoreType.DMA,
        ],
        compiler_params=pltpu.CompilerParams(use_tc_tiling_on_sc=False),
    )
    def agg_kernel(hp_hbm, src_hbm, dst_hbm, zeros_hbm, out_hbm,
                   src_v, dst_v, msg0, msg1, acc, gs0, gs1):
        c = lax.axis_index("c")
        s = lax.axis_index("s")
        w = s * NC + c
        rpz = np_ // NS
        r0 = s * rpz
        pltpu.sync_copy(src_hbm.at[pl.ds(w * rows_w, rows_w)], src_v)
        pltpu.sync_copy(dst_hbm.at[pl.ds(w * rows_w, rows_w)], dst_v)
        pltpu.async_copy(hp_hbm.at[src_v.at[0]], msg0, gs0)
        pltpu.async_copy(hp_hbm.at[src_v.at[1]], msg1, gs1)
        pltpu.sync_copy(zeros_hbm.at[pl.ds(r0, rpz)], acc.at[pl.ds(r0, rpz)])
        plsc.subcore_barrier()

        @pl.loop(0, rows_w, step=2)
        def _(j):
            pltpu.make_async_copy(hp_hbm.at[src_v.at[j]], msg0, gs0).wait()
            pltpu.sync_copy(msg0, acc.at[dst_v.at[j]], add=True)

            @pl.when(j + 2 < rows_w)
            def _():
                pltpu.async_copy(hp_hbm.at[src_v.at[j + 2]], msg0, gs0)

            pltpu.make_async_copy(hp_hbm.at[src_v.at[j + 1]], msg1, gs1).wait()
            pltpu.sync_copy(msg1, acc.at[dst_v.at[j + 1]], add=True)

            @pl.when(j + 3 < rows_w)
            def _():
                pltpu.async_copy(hp_hbm.at[src_v.at[j + 3]], msg1, gs1)

        plsc.subcore_barrier()
        pltpu.sync_copy(acc.at[pl.ds(r0, rpz)], out_hbm.at[c, pl.ds(r0, rpz)])

    return agg_kernel


def _matmul(x, w, br):
    n, d = x.shape
    h = w.shape[1]

    def body(x_ref, w_ref, o_ref):
        o_ref[...] = lax.dot_general(
            x_ref[...], w_ref[...], (((1,), (0,)), ((), ())),
            precision=_PREC, preferred_element_type=jnp.float32)

    return pl.pallas_call(
        body,
        grid=(n // br,),
        in_specs=[
            pl.BlockSpec((br, d), lambda i: (i, 0)),
            pl.BlockSpec((d, h), lambda i: (0, 0)),
        ],
        out_specs=pl.BlockSpec((br, h), lambda i: (i, 0)),
        out_shape=jax.ShapeDtypeStruct((n, h), jnp.float32),
    )(x, w)


def _deg_stats(dp_ref):
    deg = 1.0 + dp_ref[0, :, 0:1] + dp_ref[1, :, 0:1]
    return lax.rsqrt(deg), 1.0 / deg


def _scale(deg_parts, h1, br, wcols):
    n, h = h1.shape

    def body(dp_ref, h_ref, o_ref):
        dis, _ = _deg_stats(dp_ref)
        o_ref[...] = h_ref[...] * dis

    return pl.pallas_call(
        body,
        grid=(n // br,),
        in_specs=[
            pl.BlockSpec((NC, br, wcols), lambda i: (0, i, 0)),
            pl.BlockSpec((br, h), lambda i: (i, 0)),
        ],
        out_specs=pl.BlockSpec((br, h), lambda i: (i, 0)),
        out_shape=jax.ShapeDtypeStruct((n, h), jnp.float32),
    )(deg_parts, h1)


def _layer2(parts1, deg_parts, h1, b1, w2, br, wcols):
    n, h = h1.shape
    h2w = w2.shape[1]

    def body(p_ref, dp_ref, h1_ref, b1_ref, w2_ref, h2_ref, hp2_ref):
        dis, invd = _deg_stats(dp_ref)
        out1 = dis * (p_ref[0] + p_ref[1]) + h1_ref[...] * invd + b1_ref[...]
        a1 = jnp.maximum(out1, 0.0)
        h2 = lax.dot_general(a1, w2_ref[...], (((1,), (0,)), ((), ())),
                             precision=_PREC, preferred_element_type=jnp.float32)
        h2_ref[...] = h2
        hp2_ref[...] = h2 * dis

    return pl.pallas_call(
        body,
        grid=(n // br,),
        in_specs=[
            pl.BlockSpec((NC, br, h), lambda i: (0, i, 0)),
            pl.BlockSpec((NC, br, wcols), lambda i: (0, i, 0)),
            pl.BlockSpec((br, h), lambda i: (i, 0)),
            pl.BlockSpec((1, h), lambda i: (0, 0)),
            pl.BlockSpec((h, h2w), lambda i: (0, 0)),
        ],
        out_specs=[
            pl.BlockSpec((br, h2w), lambda i: (i, 0)),
            pl.BlockSpec((br, h2w), lambda i: (i, 0)),
        ],
        out_shape=[
            jax.ShapeDtypeStruct((n, h2w), jnp.float32),
            jax.ShapeDtypeStruct((n, h2w), jnp.float32),
        ],
    )(parts1, deg_parts, h1, b1, w2)


def _final(parts2, deg_parts, h2, b2, br, wcols):
    n, h = h2.shape

    def body(p_ref, dp_ref, h2_ref, b2_ref, o_ref):
        dis, invd = _deg_stats(dp_ref)
        o_ref[...] = (dis * (p_ref[0] + p_ref[1])
                      + h2_ref[...] * invd + b2_ref[...])

    return pl.pallas_call(
        body,
        grid=(n // br,),
        in_specs=[
            pl.BlockSpec((NC, br, h), lambda i: (0, i, 0)),
            pl.BlockSpec((NC, br, wcols), lambda i: (0, i, 0)),
            pl.BlockSpec((br, h), lambda i: (i, 0)),
            pl.BlockSpec((1, h), lambda i: (0, 0)),
        ],
        out_specs=pl.BlockSpec((br, h), lambda i: (i, 0)),
        out_shape=jax.ShapeDtypeStruct((n, h), jnp.float32),
    )(parts2, deg_parts, h2, b2)


def kernel(x, edge_index, W1, b1, W2, b2):
    n, d = x.shape
    e = edge_index.shape[1]
    h1w = W1.shape[1]
    h2w = W2.shape[1]

    br = 1024
    np_ = ((n + br - 1) // br) * br
    wcols = 16
    rows_w = -(-e // (CH * NW))
    rows_w = ((rows_w + 7) // 8) * 8
    e_pad = rows_w * NW * CH

    sent = jnp.int32(n)
    pad = jnp.full((e_pad - e,), sent, jnp.int32)
    src2d = jnp.concatenate([edge_index[0], pad]).reshape(NW * rows_w, CH)
    dst2d = jnp.concatenate([edge_index[1], pad]).reshape(NW * rows_w, CH)

    x_pad = jnp.pad(x, ((0, np_ - n), (0, 0)))
    ones_img = jnp.ones((CH, wcols), jnp.float32)
    zeros_w = jnp.zeros((np_, wcols), jnp.float32)
    zeros_h1 = jnp.zeros((np_, h1w), jnp.float32)
    zeros_h2 = jnp.zeros((np_, h2w), jnp.float32)

    deg_parts = _deg_kernel_factory(rows_w, np_, wcols)(dst2d, ones_img, zeros_w)
    h1 = _matmul(x_pad, W1, br)
    hp1 = _scale(deg_parts, h1, br, wcols)
    parts1 = _agg_kernel_factory(rows_w, np_, h1w)(hp1, src2d, dst2d, zeros_h1)
    h2, hp2 = _layer2(parts1, deg_parts, h1, b1.reshape(1, h1w), W2, br, wcols)
    parts2 = _agg_kernel_factory(rows_w, np_, h2w)(hp2, src2d, dst2d, zeros_h2)
    out = _final(parts2, deg_parts, h2, b2.reshape(1, h2w), br, wcols)
    return out[:n]
